# Initial kernel scaffold; baseline (speedup 1.0000x reference)
import jax
import jax.numpy as jnp
from jax.experimental import pallas as pl


def kernel(x, edge_index, edge_type, W_self, b_self, W, att_src, att_dst, b):
    raise NotImplementedError("write your pallas kernel here")


# speedup vs baseline: 76.1465x; 76.1465x over previous
import functools

import jax
import jax.numpy as jnp
from jax import lax
from jax.experimental import pallas as pl
from jax.experimental.pallas import tpu as pltpu
from jax.experimental.pallas import tpu_sc as plsc

NC = 2
NS = 16
NW = NC * NS
LANES = 16
BLK = 128
NEG = -1e38


def _dense_body(x_ref, w_ref, as_ref, ad_ref, xl_ref, a_s_ref, a_d_ref, easf_ref):
    xl = jnp.dot(x_ref[...], w_ref[0], preferred_element_type=jnp.float32)
    xl_ref[0] = xl
    a_s = jnp.sum(xl * as_ref[0, 0][None, :], axis=1)
    a_d = jnp.sum(xl * ad_ref[0, 0][None, :], axis=1)
    a_s_ref[0, 0] = a_s
    a_d_ref[0, 0] = a_d
    al = a_s + a_d
    al = jnp.where(al > 0, al, 0.2 * al)
    easf_ref[0, 0] = jnp.exp(al)


def _dense_stage(x, W, att_src, att_dst):
    N, D = x.shape
    R = W.shape[0]
    return pl.pallas_call(
        _dense_body,
        grid=(R,),
        in_specs=[
            pl.BlockSpec((N, D), lambda r: (0, 0)),
            pl.BlockSpec((1, D, D), lambda r: (r, 0, 0)),
            pl.BlockSpec((1, 1, D), lambda r: (r, 0, 0)),
            pl.BlockSpec((1, 1, D), lambda r: (r, 0, 0)),
        ],
        out_specs=[
            pl.BlockSpec((1, N, D), lambda r: (r, 0, 0)),
            pl.BlockSpec((1, 1, N), lambda r: (r, 0, 0)),
            pl.BlockSpec((1, 1, N), lambda r: (r, 0, 0)),
            pl.BlockSpec((1, 1, N), lambda r: (r, 0, 0)),
        ],
        out_shape=[
            jax.ShapeDtypeStruct((R, N, D), jnp.float32),
            jax.ShapeDtypeStruct((R, 1, N), jnp.float32),
            jax.ShapeDtypeStruct((R, 1, N), jnp.float32),
            jax.ShapeDtypeStruct((R, 1, N), jnp.float32),
        ],
    )(x, W, att_src.reshape(R, 1, D), att_dst.reshape(R, 1, D))


def _make_kernel_a(E_pad, DN, ept):
    nblk = ept // BLK
    stripe = DN // NS
    mesh = plsc.VectorSubcoreMesh(core_axis_name="c", subcore_axis_name="s", num_cores=NC, num_subcores=NS)

    @functools.partial(
        pl.kernel,
        out_type=[
            jax.ShapeDtypeStruct((E_pad,), jnp.float32),
            jax.ShapeDtypeStruct((DN,), jnp.float32),
            jax.ShapeDtypeStruct((DN,), jnp.float32),
        ],
        mesh=mesh,
        scratch_types=[
            pltpu.VMEM((BLK,), jnp.int32),
            pltpu.VMEM((BLK,), jnp.int32),
            pltpu.VMEM((BLK,), jnp.float32),
            pltpu.VMEM((BLK,), jnp.float32),
            pltpu.VMEM((BLK,), jnp.float32),
            pltpu.VMEM((stripe,), jnp.float32),
            pltpu.VMEM_SHARED((DN,), jnp.float32),
            pltpu.SemaphoreType.DMA,
            pltpu.SemaphoreType.DMA,
        ],
    )
    def ka(fs_hbm, fd_hbm, asf_hbm, adf_hbm, easf_hbm,
           ee_hbm, den0_hbm, den1_hbm,
           fs_b, fd_b, as_b, ad_b, ee_b, zb, den_sh, sem1, sem2):
        c = lax.axis_index("c")
        t = lax.axis_index("s")
        wid = t * NC + c

        @pl.when(c == 0)
        def _():
            pltpu.sync_copy(easf_hbm.at[pl.ds(t * stripe, stripe)], zb)

        @pl.when(c != 0)
        def _():
            def _zb(i, _):
                zb[pl.ds(i * LANES, LANES)] = jnp.zeros((LANES,), jnp.float32)
                return 0
            lax.fori_loop(0, stripe // LANES, _zb, 0)

        pltpu.sync_copy(zb, den_sh.at[pl.ds(t * stripe, stripe)])
        plsc.subcore_barrier()

        def blk_body(i, _):
            base = wid * ept + i * BLK
            pltpu.sync_copy(fs_hbm.at[pl.ds(base, BLK)], fs_b)
            pltpu.sync_copy(fd_hbm.at[pl.ds(base, BLK)], fd_b)
            pltpu.async_copy(asf_hbm.at[fs_b], as_b, sem1).wait()
            pltpu.async_copy(adf_hbm.at[fd_b], ad_b, sem2).wait()
            for k in range(BLK // LANES):
                sl = pl.ds(k * LANES, LANES)
                a = as_b[sl] + ad_b[sl]
                a = jnp.where(a > 0, a, 0.2 * a)
                ee_b[sl] = jnp.exp(a)
            pltpu.sync_copy(ee_b, ee_hbm.at[pl.ds(base, BLK)])
            pltpu.sync_copy(ee_b, den_sh.at[fd_b], add=True)
            return 0

        lax.fori_loop(0, nblk, blk_body, 0)
        plsc.subcore_barrier()

        pltpu.sync_copy(den_sh.at[pl.ds(t * stripe, stripe)], zb)

        @pl.when(c == 0)
        def _():
            pltpu.sync_copy(zb, den0_hbm.at[pl.ds(t * stripe, stripe)])

        @pl.when(c != 0)
        def _():
            pltpu.sync_copy(zb, den1_hbm.at[pl.ds(t * stripe, stripe)])

    return ka


def _make_kernel_b(N, D, E_pad, DN, ept):
    nblk = ept // BLK
    rchunk = 80
    nchunk = N // rchunk
    cpt = -(-nchunk // NS)
    mesh = plsc.VectorSubcoreMesh(core_axis_name="c", subcore_axis_name="s", num_cores=NC, num_subcores=NS)

    @functools.partial(
        pl.kernel,
        out_type=jax.ShapeDtypeStruct((NC, N, D), jnp.float32),
        mesh=mesh,
        scratch_types=[
            pltpu.VMEM((BLK,), jnp.int32),
            pltpu.VMEM((BLK,), jnp.int32),
            pltpu.VMEM((BLK,), jnp.int32),
            pltpu.VMEM((BLK,), jnp.float32),
            pltpu.VMEM((BLK,), jnp.float32),
            pltpu.VMEM((BLK,), jnp.float32),
            pltpu.VMEM((BLK + LANES,), jnp.float32),
            pltpu.VMEM((BLK, D), jnp.float32),
            pltpu.VMEM_SHARED((N, D), jnp.float32),
            pltpu.SemaphoreType.DMA,
            pltpu.SemaphoreType.DMA,
        ],
    )
    def kb(xl_hbm, fs_hbm, fd_hbm, dst_hbm, ee_hbm, den0_hbm, den1_hbm,
           out_hbm,
           fs_b, fd_b, dst_b, ee_b, d0_b, d1_b, coef_b, rows, out_sh,
           sem1, sem2):
        c = lax.axis_index("c")
        t = lax.axis_index("s")
        wid = t * NC + c

        def _zr(j, _):
            for k in range(D // LANES):
                rows[j, pl.ds(k * LANES, LANES)] = jnp.zeros((LANES,), jnp.float32)
            return 0
        lax.fori_loop(0, BLK, _zr, 0)
        for j in range(cpt):
            cid = t + j * NS

            @pl.when(cid < nchunk)
            def _():
                r0 = pl.multiple_of(cid * rchunk, 8)
                pltpu.sync_copy(rows.at[pl.ds(0, rchunk)],
                                out_sh.at[pl.ds(r0, rchunk)])
        plsc.subcore_barrier()

        def blk_body(i, _):
            base = wid * ept + i * BLK
            pltpu.sync_copy(fs_hbm.at[pl.ds(base, BLK)], fs_b)
            pltpu.sync_copy(fd_hbm.at[pl.ds(base, BLK)], fd_b)
            pltpu.sync_copy(dst_hbm.at[pl.ds(base, BLK)], dst_b)
            pltpu.sync_copy(ee_hbm.at[pl.ds(base, BLK)], ee_b)
            pltpu.async_copy(den0_hbm.at[fd_b], d0_b, sem1).wait()
            pltpu.async_copy(den1_hbm.at[fd_b], d1_b, sem1).wait()
            pltpu.async_copy(xl_hbm.at[fs_b], rows, sem2).wait()
            for k in range(BLK // LANES):
                sl = pl.ds(k * LANES, LANES)
                coef_b[sl] = ee_b[sl] / (d0_b[sl] + d1_b[sl] + 1e-16)

            def escale(j, _):
                cf = coef_b[pl.ds(j, LANES)][0]
                for k in range(D // LANES):
                    sl = pl.ds(k * LANES, LANES)
                    rows[j, sl] = rows[j, sl] * cf
                return 0

            lax.fori_loop(0, BLK, escale, 0)
            pltpu.sync_copy(rows, out_sh.at[dst_b], add=True)
            return 0

        lax.fori_loop(0, nblk, blk_body, 0)
        plsc.subcore_barrier()
        for j in range(cpt):
            cid = t + j * NS

            @pl.when(cid < nchunk)
            def _():
                r0 = pl.multiple_of(cid * rchunk, 8)
                pltpu.sync_copy(out_sh.at[pl.ds(r0, rchunk)],
                                rows.at[pl.ds(0, rchunk)])
                pltpu.sync_copy(rows.at[pl.ds(0, rchunk)],
                                out_hbm.at[c, pl.ds(r0, rchunk)])

    return kb


def _combine_body(x_ref, ws_ref, bs_ref, b_ref, p_ref, xl_ref,
                  easf_ref, d0_ref, d1_ref, o_ref):
    R = b_ref.shape[0]
    acc = jnp.dot(x_ref[...], ws_ref[...], preferred_element_type=jnp.float32)
    acc += (bs_ref[0] + jnp.sum(b_ref[...], axis=0))[None, :]
    acc += p_ref[0] + p_ref[1]
    cs = easf_ref[...] / (d0_ref[...] + d1_ref[...] + 1e-16)
    for r in range(R):
        acc += xl_ref[r] * cs[:, r][:, None]
    o_ref[...] = acc


def _combine_stage(x, W_self, b_self, b, partials, xl, easf_t, d0_t, d1_t):
    N, D = x.shape
    R = b.shape[0]
    BN = 2000
    nb = N // BN
    return pl.pallas_call(
        _combine_body,
        grid=(nb,),
        in_specs=[
            pl.BlockSpec((BN, D), lambda i: (i, 0)),
            pl.BlockSpec((D, D), lambda i: (0, 0)),
            pl.BlockSpec((1, D), lambda i: (0, 0)),
            pl.BlockSpec((R, D), lambda i: (0, 0)),
            pl.BlockSpec((NC, BN, D), lambda i: (0, i, 0)),
            pl.BlockSpec((R, BN, D), lambda i: (0, i, 0)),
            pl.BlockSpec((BN, R), lambda i: (i, 0)),
            pl.BlockSpec((BN, R), lambda i: (i, 0)),
            pl.BlockSpec((BN, R), lambda i: (i, 0)),
        ],
        out_specs=pl.BlockSpec((BN, D), lambda i: (i, 0)),
        out_shape=jax.ShapeDtypeStruct((N, D), jnp.float32),
    )(x, W_self, b_self.reshape(1, D), b, partials, xl, easf_t, d0_t, d1_t)


def kernel(x, edge_index, edge_type, W_self, b_self, W, att_src, att_dst, b):
    N, D = x.shape
    R = W.shape[0]
    E = edge_index.shape[1]
    RN = R * N
    DN = RN + 128

    ept = -(-(-(-E // NW) // BLK)) * BLK
    ept = ((E + NW - 1) // NW + BLK - 1) // BLK * BLK
    E_pad = ept * NW
    pad = E_pad - E

    src = edge_index[0].astype(jnp.int32)
    dst = edge_index[1].astype(jnp.int32)
    et = edge_type.astype(jnp.int32)
    fs = et * N + src
    fd = et * N + dst
    fs_p = jnp.pad(fs, (0, pad), constant_values=0)
    fd_p = jnp.pad(fd, (0, pad), constant_values=RN)
    dst_p = jnp.pad(dst, (0, pad), constant_values=0)

    xl, a_s, a_d, easf = _dense_stage(x, W, att_src, att_dst)

    neg = jnp.full((8,), NEG, jnp.float32)
    asf_ext = jnp.concatenate([a_s.reshape(RN), neg])
    adf_ext = jnp.concatenate([a_d.reshape(RN), neg])
    easf_pad = jnp.concatenate([easf.reshape(RN),
                                jnp.zeros((DN - RN,), jnp.float32)])

    ka = _make_kernel_a(E_pad, DN, ept)
    ee, den0, den1 = ka(fs_p, fd_p, asf_ext, adf_ext, easf_pad)

    kb = _make_kernel_b(N, D, E_pad, DN, ept)
    partials = kb(xl.reshape(RN, D), fs_p, fd_p, dst_p, ee, den0, den1)

    easf_t = easf.reshape(R, N).T
    d0_t = den0[:RN].reshape(R, N).T
    d1_t = den1[:RN].reshape(R, N).T
    return _combine_stage(x, W_self, b_self, b, partials, xl, easf_t, d0_t, d1_t)


# speedup vs baseline: 82.0987x; 1.0782x over previous
import functools

import jax
import jax.numpy as jnp
from jax import lax
from jax.experimental import pallas as pl
from jax.experimental.pallas import tpu as pltpu
from jax.experimental.pallas import tpu_sc as plsc

NC = 2
NS = 16
NW = NC * NS
LANES = 16
BLK = 128
NEG = -1e38


def _dense_body(x_ref, w_ref, as_ref, ad_ref, xl_ref, a_s_ref, a_d_ref, easf_ref):
    xl = jnp.dot(x_ref[...], w_ref[0], preferred_element_type=jnp.float32)
    xl_ref[0] = xl
    a_s = jnp.sum(xl * as_ref[0, 0][None, :], axis=1)
    a_d = jnp.sum(xl * ad_ref[0, 0][None, :], axis=1)
    a_s_ref[0, 0] = a_s
    a_d_ref[0, 0] = a_d
    al = a_s + a_d
    al = jnp.where(al > 0, al, 0.2 * al)
    easf_ref[0, 0] = jnp.exp(al)


def _dense_stage(x, W, att_src, att_dst):
    N, D = x.shape
    R = W.shape[0]
    return pl.pallas_call(
        _dense_body,
        grid=(R,),
        in_specs=[
            pl.BlockSpec((N, D), lambda r: (0, 0)),
            pl.BlockSpec((1, D, D), lambda r: (r, 0, 0)),
            pl.BlockSpec((1, 1, D), lambda r: (r, 0, 0)),
            pl.BlockSpec((1, 1, D), lambda r: (r, 0, 0)),
        ],
        out_specs=[
            pl.BlockSpec((1, N, D), lambda r: (r, 0, 0)),
            pl.BlockSpec((1, 1, N), lambda r: (r, 0, 0)),
            pl.BlockSpec((1, 1, N), lambda r: (r, 0, 0)),
            pl.BlockSpec((1, 1, N), lambda r: (r, 0, 0)),
        ],
        out_shape=[
            jax.ShapeDtypeStruct((R, N, D), jnp.float32),
            jax.ShapeDtypeStruct((R, 1, N), jnp.float32),
            jax.ShapeDtypeStruct((R, 1, N), jnp.float32),
            jax.ShapeDtypeStruct((R, 1, N), jnp.float32),
        ],
    )(x, W, att_src.reshape(R, 1, D), att_dst.reshape(R, 1, D))


def _make_kernel_a(E_pad, DN, ept):
    nblk = ept // BLK
    stripe = DN // NS
    mesh = plsc.VectorSubcoreMesh(core_axis_name="c", subcore_axis_name="s", num_cores=NC, num_subcores=NS)

    @functools.partial(
        pl.kernel,
        out_type=[
            jax.ShapeDtypeStruct((E_pad,), jnp.float32),
            jax.ShapeDtypeStruct((DN,), jnp.float32),
            jax.ShapeDtypeStruct((DN,), jnp.float32),
        ],
        mesh=mesh,
        scratch_types=[
            [pltpu.VMEM((BLK,), jnp.int32)] * 2,
            [pltpu.VMEM((BLK,), jnp.int32)] * 2,
            [pltpu.VMEM((BLK,), jnp.float32)] * 2,
            [pltpu.VMEM((BLK,), jnp.float32)] * 2,
            [pltpu.VMEM((BLK,), jnp.float32)] * 2,
            pltpu.VMEM((stripe,), jnp.float32),
            pltpu.VMEM_SHARED((DN,), jnp.float32),
            [[pltpu.SemaphoreType.DMA] * 2 for _ in range(6)],
        ],
    )
    def ka(fs_hbm, fd_hbm, asf_hbm, adf_hbm, easf_hbm,
           ee_hbm, den0_hbm, den1_hbm,
           fs_b, fd_b, as_b, ad_b, ee_b, zb, den_sh, sem):
        c = lax.axis_index("c")
        t = lax.axis_index("s")
        wid = t * NC + c

        @pl.when(c == 0)
        def _():
            pltpu.sync_copy(easf_hbm.at[pl.ds(t * stripe, stripe)], zb)

        @pl.when(c != 0)
        def _():
            def _zb(i, _):
                zb[pl.ds(i * LANES, LANES)] = jnp.zeros((LANES,), jnp.float32)
                return 0
            lax.fori_loop(0, stripe // LANES, _zb, 0)

        pltpu.sync_copy(zb, den_sh.at[pl.ds(t * stripe, stripe)])
        plsc.subcore_barrier()

        base0 = wid * ept
        pltpu.sync_copy(fs_hbm.at[pl.ds(base0, BLK)], fs_b[0])
        pltpu.sync_copy(fd_hbm.at[pl.ds(base0, BLK)], fd_b[0])
        pltpu.async_copy(asf_hbm.at[fs_b[0]], as_b[0], sem[2][0]).wait()
        pltpu.async_copy(adf_hbm.at[fd_b[0]], ad_b[0], sem[3][0]).wait()

        def half(i, p):
            q = 1 - p
            nbase = base0 + jnp.minimum(i + 1, nblk - 1) * BLK
            la = pltpu.async_copy(fs_hbm.at[pl.ds(nbase, BLK)], fs_b[q], sem[0][q])
            lb = pltpu.async_copy(fd_hbm.at[pl.ds(nbase, BLK)], fd_b[q], sem[1][q])
            for k in range(BLK // LANES):
                slc = pl.ds(k * LANES, LANES)
                a = as_b[p][slc] + ad_b[p][slc]
                a = jnp.where(a > 0, a, 0.2 * a)
                ee_b[p][slc] = jnp.exp(a)
            la.wait()
            lb.wait()
            ga = pltpu.async_copy(asf_hbm.at[fs_b[q]], as_b[q], sem[2][q])
            gb = pltpu.async_copy(adf_hbm.at[fd_b[q]], ad_b[q], sem[3][q])
            base = base0 + i * BLK
            st = pltpu.async_copy(ee_b[p], ee_hbm.at[pl.ds(base, BLK)], sem[4][p])
            sc = pltpu.async_copy(ee_b[p], den_sh.at[fd_b[p]], sem[5][p], add=True)
            ga.wait()
            gb.wait()
            st.wait()
            sc.wait()

        def g_body(g, _):
            half(2 * g, 0)
            half(2 * g + 1, 1)
            return 0

        lax.fori_loop(0, nblk // 2, g_body, 0)
        plsc.subcore_barrier()

        pltpu.sync_copy(den_sh.at[pl.ds(t * stripe, stripe)], zb)

        @pl.when(c == 0)
        def _():
            pltpu.sync_copy(zb, den0_hbm.at[pl.ds(t * stripe, stripe)])

        @pl.when(c != 0)
        def _():
            pltpu.sync_copy(zb, den1_hbm.at[pl.ds(t * stripe, stripe)])

    return ka


def _make_kernel_b(N, D, E_pad, DN, ept):
    nblk = ept // BLK
    rchunk = 80
    nchunk = N // rchunk
    cpt = -(-nchunk // NS)
    mesh = plsc.VectorSubcoreMesh(core_axis_name="c", subcore_axis_name="s", num_cores=NC, num_subcores=NS)

    @functools.partial(
        pl.kernel,
        out_type=jax.ShapeDtypeStruct((NC, N, D), jnp.float32),
        mesh=mesh,
        scratch_types=[
            [pltpu.VMEM((BLK,), jnp.int32)] * 2,
            [pltpu.VMEM((BLK,), jnp.int32)] * 2,
            [pltpu.VMEM((BLK,), jnp.int32)] * 2,
            [pltpu.VMEM((BLK,), jnp.float32)] * 2,
            [pltpu.VMEM((BLK,), jnp.float32)] * 2,
            [pltpu.VMEM((BLK,), jnp.float32)] * 2,
            [pltpu.VMEM((BLK + LANES,), jnp.float32)] * 2,
            [pltpu.VMEM((BLK, D), jnp.float32)] * 2,
            pltpu.VMEM_SHARED((N, D), jnp.float32),
            [[pltpu.SemaphoreType.DMA] * 2 for _ in range(8)],
        ],
    )
    def kb(xl_hbm, fs_hbm, fd_hbm, dst_hbm, ee_hbm, den0_hbm, den1_hbm,
           out_hbm,
           fs_b, fd_b, dst_b, ee_b, d0_b, d1_b, coef_b, rows, out_sh,
           sem):
        c = lax.axis_index("c")
        t = lax.axis_index("s")
        wid = t * NC + c

        def _zr(j, _):
            for k in range(D // LANES):
                rows[0][j, pl.ds(k * LANES, LANES)] = jnp.zeros((LANES,), jnp.float32)
            return 0
        lax.fori_loop(0, BLK, _zr, 0)
        for j in range(cpt):
            cid = t + j * NS

            @pl.when(cid < nchunk)
            def _():
                r0 = pl.multiple_of(cid * rchunk, 8)
                pltpu.sync_copy(rows[0].at[pl.ds(0, rchunk)],
                                out_sh.at[pl.ds(r0, rchunk)])
        plsc.subcore_barrier()

        base0 = wid * ept

        def issue_lin(i, p):
            base = base0 + i * BLK
            return (
                pltpu.async_copy(fs_hbm.at[pl.ds(base, BLK)], fs_b[p], sem[0][p]),
                pltpu.async_copy(fd_hbm.at[pl.ds(base, BLK)], fd_b[p], sem[1][p]),
                pltpu.async_copy(dst_hbm.at[pl.ds(base, BLK)], dst_b[p], sem[2][p]),
                pltpu.async_copy(ee_hbm.at[pl.ds(base, BLK)], ee_b[p], sem[3][p]),
            )

        def issue_ind(p):
            return (
                pltpu.async_copy(den0_hbm.at[fd_b[p]], d0_b[p], sem[4][p]),
                pltpu.async_copy(den1_hbm.at[fd_b[p]], d1_b[p], sem[5][p]),
                pltpu.async_copy(xl_hbm.at[fs_b[p]], rows[p], sem[6][p]),
            )

        for d in issue_lin(0, 0):
            d.wait()
        for d in issue_ind(0):
            d.wait()

        def half(i, p):
            q = 1 - p
            nxt = jnp.minimum(i + 1, nblk - 1)
            lds = issue_lin(nxt, q)
            for k in range(BLK // LANES):
                slc = pl.ds(k * LANES, LANES)
                coef_b[p][slc] = ee_b[p][slc] / (d0_b[p][slc] + d1_b[p][slc] + 1e-16)

            def escale(j, _):
                cf = coef_b[p][pl.ds(j, LANES)][0]
                for k in range(D // LANES):
                    slc = pl.ds(k * LANES, LANES)
                    rows[p][j, slc] = rows[p][j, slc] * cf
                return 0

            lax.fori_loop(0, BLK, escale, 0)
            for d in lds:
                d.wait()
            gds = issue_ind(q)
            sc = pltpu.async_copy(rows[p], out_sh.at[dst_b[p]], sem[7][p],
                                  add=True)
            for d in gds:
                d.wait()
            sc.wait()

        def g_body(g, _):
            half(2 * g, 0)
            half(2 * g + 1, 1)
            return 0

        lax.fori_loop(0, nblk // 2, g_body, 0)
        plsc.subcore_barrier()
        for j in range(cpt):
            cid = t + j * NS

            @pl.when(cid < nchunk)
            def _():
                r0 = pl.multiple_of(cid * rchunk, 8)
                pltpu.sync_copy(out_sh.at[pl.ds(r0, rchunk)],
                                rows[0].at[pl.ds(0, rchunk)])
                pltpu.sync_copy(rows[0].at[pl.ds(0, rchunk)],
                                out_hbm.at[c, pl.ds(r0, rchunk)])

    return kb


def _combine_body(x_ref, ws_ref, bs_ref, b_ref, p_ref, xl_ref,
                  easf_ref, d0_ref, d1_ref, o_ref):
    R = b_ref.shape[0]
    acc = jnp.dot(x_ref[...], ws_ref[...], preferred_element_type=jnp.float32)
    acc += (bs_ref[0] + jnp.sum(b_ref[...], axis=0))[None, :]
    acc += p_ref[0] + p_ref[1]
    cs = easf_ref[...] / (d0_ref[...] + d1_ref[...] + 1e-16)
    for r in range(R):
        acc += xl_ref[r] * cs[:, r][:, None]
    o_ref[...] = acc


def _combine_stage(x, W_self, b_self, b, partials, xl, easf_t, d0_t, d1_t):
    N, D = x.shape
    R = b.shape[0]
    BN = 2000
    nb = N // BN
    return pl.pallas_call(
        _combine_body,
        grid=(nb,),
        in_specs=[
            pl.BlockSpec((BN, D), lambda i: (i, 0)),
            pl.BlockSpec((D, D), lambda i: (0, 0)),
            pl.BlockSpec((1, D), lambda i: (0, 0)),
            pl.BlockSpec((R, D), lambda i: (0, 0)),
            pl.BlockSpec((NC, BN, D), lambda i: (0, i, 0)),
            pl.BlockSpec((R, BN, D), lambda i: (0, i, 0)),
            pl.BlockSpec((BN, R), lambda i: (i, 0)),
            pl.BlockSpec((BN, R), lambda i: (i, 0)),
            pl.BlockSpec((BN, R), lambda i: (i, 0)),
        ],
        out_specs=pl.BlockSpec((BN, D), lambda i: (i, 0)),
        out_shape=jax.ShapeDtypeStruct((N, D), jnp.float32),
    )(x, W_self, b_self.reshape(1, D), b, partials, xl, easf_t, d0_t, d1_t)


def kernel(x, edge_index, edge_type, W_self, b_self, W, att_src, att_dst, b):
    N, D = x.shape
    R = W.shape[0]
    E = edge_index.shape[1]
    RN = R * N
    DN = RN + 128

    ept = ((E + NW - 1) // NW + 2 * BLK - 1) // (2 * BLK) * (2 * BLK)
    E_pad = ept * NW
    pad = E_pad - E

    src = edge_index[0].astype(jnp.int32)
    dst = edge_index[1].astype(jnp.int32)
    et = edge_type.astype(jnp.int32)
    fs = et * N + src
    fd = et * N + dst
    fs_p = jnp.pad(fs, (0, pad), constant_values=0)
    fd_p = jnp.pad(fd, (0, pad), constant_values=RN)
    dst_p = jnp.pad(dst, (0, pad), constant_values=0)

    xl, a_s, a_d, easf = _dense_stage(x, W, att_src, att_dst)

    neg = jnp.full((8,), NEG, jnp.float32)
    asf_ext = jnp.concatenate([a_s.reshape(RN), neg])
    adf_ext = jnp.concatenate([a_d.reshape(RN), neg])
    easf_pad = jnp.concatenate([easf.reshape(RN),
                                jnp.zeros((DN - RN,), jnp.float32)])

    ka = _make_kernel_a(E_pad, DN, ept)
    ee, den0, den1 = ka(fs_p, fd_p, asf_ext, adf_ext, easf_pad)

    kb = _make_kernel_b(N, D, E_pad, DN, ept)
    partials = kb(xl.reshape(RN, D), fs_p, fd_p, dst_p, ee, den0, den1)

    easf_t = easf.reshape(R, N).T
    d0_t = den0[:RN].reshape(R, N).T
    d1_t = den1[:RN].reshape(R, N).T
    return _combine_stage(x, W_self, b_self, b, partials, xl, easf_t, d0_t, d1_t)


# speedup vs baseline: 85.3043x; 1.0390x over previous
import functools

import jax
import jax.numpy as jnp
from jax import lax
from jax.experimental import pallas as pl
from jax.experimental.pallas import tpu as pltpu
from jax.experimental.pallas import tpu_sc as plsc

NC = 2
NS = 16
NW = NC * NS
LANES = 16
BLK = 128
NEG = -1e38


def _alpha_body(x_ref, w_ref, as_ref, ad_ref, a_s_ref, a_d_ref, easf_ref):
    ws = jnp.dot(w_ref[0], as_ref[0, 0][:, None],
                 preferred_element_type=jnp.float32)[:, 0]
    wd = jnp.dot(w_ref[0], ad_ref[0, 0][:, None],
                 preferred_element_type=jnp.float32)[:, 0]
    a_s = jnp.sum(x_ref[...] * ws[None, :], axis=1)
    a_d = jnp.sum(x_ref[...] * wd[None, :], axis=1)
    a_s_ref[0, 0] = a_s
    a_d_ref[0, 0] = a_d
    al = a_s + a_d
    al = jnp.where(al > 0, al, 0.2 * al)
    easf_ref[0, 0] = jnp.exp(al)


def _alpha_stage(x, W, att_src, att_dst):
    N, D = x.shape
    R = W.shape[0]
    return pl.pallas_call(
        _alpha_body,
        grid=(R,),
        in_specs=[
            pl.BlockSpec((N, D), lambda r: (0, 0)),
            pl.BlockSpec((1, D, D), lambda r: (r, 0, 0)),
            pl.BlockSpec((1, 1, D), lambda r: (r, 0, 0)),
            pl.BlockSpec((1, 1, D), lambda r: (r, 0, 0)),
        ],
        out_specs=[
            pl.BlockSpec((1, 1, N), lambda r: (r, 0, 0)),
            pl.BlockSpec((1, 1, N), lambda r: (r, 0, 0)),
            pl.BlockSpec((1, 1, N), lambda r: (r, 0, 0)),
        ],
        out_shape=[
            jax.ShapeDtypeStruct((R, 1, N), jnp.float32),
            jax.ShapeDtypeStruct((R, 1, N), jnp.float32),
            jax.ShapeDtypeStruct((R, 1, N), jnp.float32),
        ],
    )(x, W, att_src.reshape(R, 1, D), att_dst.reshape(R, 1, D))


def _xl_body(x_ref, w_ref, xl_ref):
    xl_ref[0] = jnp.dot(x_ref[...], w_ref[0], preferred_element_type=jnp.float32)


def _dense_stage(x, W):
    N, D = x.shape
    R = W.shape[0]
    return pl.pallas_call(
        _xl_body,
        grid=(R,),
        in_specs=[
            pl.BlockSpec((N, D), lambda r: (0, 0)),
            pl.BlockSpec((1, D, D), lambda r: (r, 0, 0)),
        ],
        out_specs=pl.BlockSpec((1, N, D), lambda r: (r, 0, 0)),
        out_shape=jax.ShapeDtypeStruct((R, N, D), jnp.float32),
    )(x, W)


def _make_kernel_a(E_pad, DN, ept):
    nblk = ept // BLK
    stripe = DN // NS
    mesh = plsc.VectorSubcoreMesh(core_axis_name="c", subcore_axis_name="s", num_cores=NC, num_subcores=NS)

    @functools.partial(
        pl.kernel,
        out_type=[
            jax.ShapeDtypeStruct((E_pad,), jnp.float32),
            jax.ShapeDtypeStruct((DN,), jnp.float32),
            jax.ShapeDtypeStruct((DN,), jnp.float32),
        ],
        mesh=mesh,
        scratch_types=[
            [pltpu.VMEM((BLK,), jnp.int32)] * 2,
            [pltpu.VMEM((BLK,), jnp.int32)] * 2,
            [pltpu.VMEM((BLK,), jnp.float32)] * 2,
            [pltpu.VMEM((BLK,), jnp.float32)] * 2,
            [pltpu.VMEM((BLK,), jnp.float32)] * 2,
            pltpu.VMEM((stripe,), jnp.float32),
            pltpu.VMEM_SHARED((DN,), jnp.float32),
            [[pltpu.SemaphoreType.DMA] * 2 for _ in range(6)],
        ],
    )
    def ka(fs_hbm, fd_hbm, asf_hbm, adf_hbm, easf_hbm,
           ee_hbm, den0_hbm, den1_hbm,
           fs_b, fd_b, as_b, ad_b, ee_b, zb, den_sh, sem):
        c = lax.axis_index("c")
        t = lax.axis_index("s")
        wid = t * NC + c

        @pl.when(c == 0)
        def _():
            pltpu.sync_copy(easf_hbm.at[pl.ds(t * stripe, stripe)], zb)

        @pl.when(c != 0)
        def _():
            def _zb(i, _):
                zb[pl.ds(i * LANES, LANES)] = jnp.zeros((LANES,), jnp.float32)
                return 0
            lax.fori_loop(0, stripe // LANES, _zb, 0)

        pltpu.sync_copy(zb, den_sh.at[pl.ds(t * stripe, stripe)])
        plsc.subcore_barrier()

        base0 = wid * ept
        pltpu.sync_copy(fs_hbm.at[pl.ds(base0, BLK)], fs_b[0])
        pltpu.sync_copy(fd_hbm.at[pl.ds(base0, BLK)], fd_b[0])
        pltpu.async_copy(asf_hbm.at[fs_b[0]], as_b[0], sem[2][0]).wait()
        pltpu.async_copy(adf_hbm.at[fd_b[0]], ad_b[0], sem[3][0]).wait()

        def half(i, p):
            q = 1 - p
            nbase = base0 + jnp.minimum(i + 1, nblk - 1) * BLK
            la = pltpu.async_copy(fs_hbm.at[pl.ds(nbase, BLK)], fs_b[q], sem[0][q])
            lb = pltpu.async_copy(fd_hbm.at[pl.ds(nbase, BLK)], fd_b[q], sem[1][q])
            for k in range(BLK // LANES):
                slc = pl.ds(k * LANES, LANES)
                a = as_b[p][slc] + ad_b[p][slc]
                a = jnp.where(a > 0, a, 0.2 * a)
                ee_b[p][slc] = jnp.exp(a)
            la.wait()
            lb.wait()
            ga = pltpu.async_copy(asf_hbm.at[fs_b[q]], as_b[q], sem[2][q])
            gb = pltpu.async_copy(adf_hbm.at[fd_b[q]], ad_b[q], sem[3][q])
            base = base0 + i * BLK
            st = pltpu.async_copy(ee_b[p], ee_hbm.at[pl.ds(base, BLK)], sem[4][p])
            sc = pltpu.async_copy(ee_b[p], den_sh.at[fd_b[p]], sem[5][p], add=True)
            ga.wait()
            gb.wait()
            st.wait()
            sc.wait()

        def g_body(g, _):
            half(2 * g, 0)
            half(2 * g + 1, 1)
            return 0

        lax.fori_loop(0, nblk // 2, g_body, 0)
        plsc.subcore_barrier()

        pltpu.sync_copy(den_sh.at[pl.ds(t * stripe, stripe)], zb)

        @pl.when(c == 0)
        def _():
            pltpu.sync_copy(zb, den0_hbm.at[pl.ds(t * stripe, stripe)])

        @pl.when(c != 0)
        def _():
            pltpu.sync_copy(zb, den1_hbm.at[pl.ds(t * stripe, stripe)])

    return ka


def _make_kernel_b(N, D, E_pad, DN, ept):
    nblk = ept // BLK
    rchunk = 80
    nchunk = N // rchunk
    cpt = -(-nchunk // NS)
    mesh = plsc.VectorSubcoreMesh(core_axis_name="c", subcore_axis_name="s", num_cores=NC, num_subcores=NS)

    @functools.partial(
        pl.kernel,
        out_type=jax.ShapeDtypeStruct((NC, N, D), jnp.float32),
        mesh=mesh,
        scratch_types=[
            [pltpu.VMEM((BLK,), jnp.int32)] * 2,
            [pltpu.VMEM((BLK,), jnp.int32)] * 2,
            [pltpu.VMEM((BLK,), jnp.int32)] * 2,
            [pltpu.VMEM((BLK,), jnp.float32)] * 2,
            [pltpu.VMEM((BLK,), jnp.float32)] * 2,
            [pltpu.VMEM((BLK,), jnp.float32)] * 2,
            [pltpu.VMEM((BLK + LANES,), jnp.float32)] * 2,
            [pltpu.VMEM((BLK, D), jnp.float32)] * 2,
            pltpu.VMEM_SHARED((N, D), jnp.float32),
            [[pltpu.SemaphoreType.DMA] * 2 for _ in range(8)],
        ],
    )
    def kb(xl_hbm, fs_hbm, fd_hbm, dst_hbm, ee_hbm, den0_hbm, den1_hbm,
           out_hbm,
           fs_b, fd_b, dst_b, ee_b, d0_b, d1_b, coef_b, rows, out_sh,
           sem):
        c = lax.axis_index("c")
        t = lax.axis_index("s")
        wid = t * NC + c

        def _zr(j, _):
            for k in range(D // LANES):
                rows[0][j, pl.ds(k * LANES, LANES)] = jnp.zeros((LANES,), jnp.float32)
            return 0
        lax.fori_loop(0, BLK, _zr, 0)
        for j in range(cpt):
            cid = t + j * NS

            @pl.when(cid < nchunk)
            def _():
                r0 = pl.multiple_of(cid * rchunk, 8)
                pltpu.sync_copy(rows[0].at[pl.ds(0, rchunk)],
                                out_sh.at[pl.ds(r0, rchunk)])
        plsc.subcore_barrier()

        base0 = wid * ept

        def issue_lin(i, p):
            base = base0 + i * BLK
            return (
                pltpu.async_copy(fs_hbm.at[pl.ds(base, BLK)], fs_b[p], sem[0][p]),
                pltpu.async_copy(fd_hbm.at[pl.ds(base, BLK)], fd_b[p], sem[1][p]),
                pltpu.async_copy(dst_hbm.at[pl.ds(base, BLK)], dst_b[p], sem[2][p]),
                pltpu.async_copy(ee_hbm.at[pl.ds(base, BLK)], ee_b[p], sem[3][p]),
            )

        def issue_ind(p):
            return (
                pltpu.async_copy(den0_hbm.at[fd_b[p]], d0_b[p], sem[4][p]),
                pltpu.async_copy(den1_hbm.at[fd_b[p]], d1_b[p], sem[5][p]),
                pltpu.async_copy(xl_hbm.at[fs_b[p]], rows[p], sem[6][p]),
            )

        for d in issue_lin(0, 0):
            d.wait()
        for d in issue_ind(0):
            d.wait()

        def half(i, p):
            q = 1 - p
            nxt = jnp.minimum(i + 1, nblk - 1)
            lds = issue_lin(nxt, q)
            for k in range(BLK // LANES):
                slc = pl.ds(k * LANES, LANES)
                coef_b[p][slc] = ee_b[p][slc] / (d0_b[p][slc] + d1_b[p][slc] + 1e-16)

            def escale(j, _):
                cf = coef_b[p][pl.ds(j, LANES)][0]
                for k in range(D // LANES):
                    slc = pl.ds(k * LANES, LANES)
                    rows[p][j, slc] = rows[p][j, slc] * cf
                return 0

            lax.fori_loop(0, BLK, escale, 0)
            for d in lds:
                d.wait()
            gds = issue_ind(q)
            sc = pltpu.async_copy(rows[p], out_sh.at[dst_b[p]], sem[7][p],
                                  add=True)
            for d in gds:
                d.wait()
            sc.wait()

        def g_body(g, _):
            half(2 * g, 0)
            half(2 * g + 1, 1)
            return 0

        lax.fori_loop(0, nblk // 2, g_body, 0)
        plsc.subcore_barrier()
        for j in range(cpt):
            cid = t + j * NS

            @pl.when(cid < nchunk)
            def _():
                r0 = pl.multiple_of(cid * rchunk, 8)
                pltpu.sync_copy(out_sh.at[pl.ds(r0, rchunk)],
                                rows[0].at[pl.ds(0, rchunk)])
                pltpu.sync_copy(rows[0].at[pl.ds(0, rchunk)],
                                out_hbm.at[c, pl.ds(r0, rchunk)])

    return kb


def _combine_body(x_ref, ws_ref, bs_ref, b_ref, p_ref, xl_ref,
                  easf_ref, d0_ref, d1_ref, o_ref):
    R = b_ref.shape[0]
    acc = jnp.dot(x_ref[...], ws_ref[...], preferred_element_type=jnp.float32)
    acc += (bs_ref[0] + jnp.sum(b_ref[...], axis=0))[None, :]
    acc += p_ref[0] + p_ref[1]
    cs = easf_ref[...] / (d0_ref[...] + d1_ref[...] + 1e-16)
    for r in range(R):
        acc += xl_ref[r] * cs[:, r][:, None]
    o_ref[...] = acc


def _combine_stage(x, W_self, b_self, b, partials, xl, easf_t, d0_t, d1_t):
    N, D = x.shape
    R = b.shape[0]
    BN = 2000
    nb = N // BN
    return pl.pallas_call(
        _combine_body,
        grid=(nb,),
        in_specs=[
            pl.BlockSpec((BN, D), lambda i: (i, 0)),
            pl.BlockSpec((D, D), lambda i: (0, 0)),
            pl.BlockSpec((1, D), lambda i: (0, 0)),
            pl.BlockSpec((R, D), lambda i: (0, 0)),
            pl.BlockSpec((NC, BN, D), lambda i: (0, i, 0)),
            pl.BlockSpec((R, BN, D), lambda i: (0, i, 0)),
            pl.BlockSpec((BN, R), lambda i: (i, 0)),
            pl.BlockSpec((BN, R), lambda i: (i, 0)),
            pl.BlockSpec((BN, R), lambda i: (i, 0)),
        ],
        out_specs=pl.BlockSpec((BN, D), lambda i: (i, 0)),
        out_shape=jax.ShapeDtypeStruct((N, D), jnp.float32),
    )(x, W_self, b_self.reshape(1, D), b, partials, xl, easf_t, d0_t, d1_t)


def kernel(x, edge_index, edge_type, W_self, b_self, W, att_src, att_dst, b):
    N, D = x.shape
    R = W.shape[0]
    E = edge_index.shape[1]
    RN = R * N
    DN = RN + 128

    ept = ((E + NW - 1) // NW + 2 * BLK - 1) // (2 * BLK) * (2 * BLK)
    E_pad = ept * NW
    pad = E_pad - E

    src = edge_index[0].astype(jnp.int32)
    dst = edge_index[1].astype(jnp.int32)
    et = edge_type.astype(jnp.int32)
    fs = et * N + src
    fd = et * N + dst
    fs_p = jnp.pad(fs, (0, pad), constant_values=0)
    fd_p = jnp.pad(fd, (0, pad), constant_values=RN)
    dst_p = jnp.pad(dst, (0, pad), constant_values=0)

    a_s, a_d, easf = _alpha_stage(x, W, att_src, att_dst)
    xl = _dense_stage(x, W)

    neg = jnp.full((8,), NEG, jnp.float32)
    asf_ext = jnp.concatenate([a_s.reshape(RN), neg])
    adf_ext = jnp.concatenate([a_d.reshape(RN), neg])
    easf_pad = jnp.concatenate([easf.reshape(RN),
                                jnp.zeros((DN - RN,), jnp.float32)])

    ka = _make_kernel_a(E_pad, DN, ept)
    ee, den0, den1 = ka(fs_p, fd_p, asf_ext, adf_ext, easf_pad)

    kb = _make_kernel_b(N, D, E_pad, DN, ept)
    partials = kb(xl.reshape(RN, D), fs_p, fd_p, dst_p, ee, den0, den1)

    easf_t = easf.reshape(R, N).T
    d0_t = den0[:RN].reshape(R, N).T
    d1_t = den1[:RN].reshape(R, N).T
    return _combine_stage(x, W_self, b_self, b, partials, xl, easf_t, d0_t, d1_t)
